# baseline (speedup 1.0000x reference)
import functools

import jax
import jax.numpy as jnp
from jax import lax
from jax.experimental import pallas as pl
from jax.experimental.pallas import tpu as pltpu
from jax.experimental.pallas import tpu_sc as plsc

_N_TOK = 4096
_N_CLU = 1024
_D = 64
_DP = 128
_TB = 2048


def _argmin_body(x_ref, e_ref, k_ref, ze_ref, et_s, esq_s):
    @pl.when(pl.program_id(0) == 0)
    def _prep():
        e = e_ref[...]
        et = e.T
        et_s[...] = et
        esq_s[...] = jnp.sum(et * et, axis=0, keepdims=True)

    x = x_ref[...]
    dots = lax.dot_general(x, et_s[...], (((1,), (0,)), ((), ())),
                           precision=lax.Precision.HIGHEST,
                           preferred_element_type=jnp.float32)
    scores = esq_s[...] - 2.0 * dots
    k = jnp.argmin(scores, axis=1).astype(jnp.int32)
    k_ref[...] = k.reshape(1, _TB // 128, 128)
    ze_ref[...] = x


def _argmin_call(x, embeddings):
    grid = _N_TOK // _TB
    return pl.pallas_call(
        _argmin_body,
        grid=(grid,),
        in_specs=[
            pl.BlockSpec((_TB, _D), lambda i: (i, 0)),
            pl.BlockSpec((_N_CLU, _D), lambda i: (0, 0)),
        ],
        out_specs=[
            pl.BlockSpec((1, _TB // 128, 128), lambda i: (i, 0, 0)),
            pl.BlockSpec((_TB, _D), lambda i: (i, 0)),
        ],
        out_shape=[
            jax.ShapeDtypeStruct((_N_TOK // _TB, _TB // 128, 128), jnp.int32),
            jax.ShapeDtypeStruct((_N_TOK, _D), jnp.float32),
        ],
        scratch_shapes=[
            pltpu.VMEM((_D, _N_CLU), jnp.float32),
            pltpu.VMEM((1, _N_CLU), jnp.float32),
        ],
    )(x, embeddings)


@functools.partial(jax.jit, static_argnames=())
def _gather_rows(table, idx):
    info = plsc.get_sparse_core_info()
    nw = info.num_subcores
    b_per_w = _N_TOK // nw
    mesh = plsc.VectorSubcoreMesh(core_axis_name="c", subcore_axis_name="s",
                                  num_cores=1)

    @functools.partial(
        pl.kernel, mesh=mesh,
        out_type=jax.ShapeDtypeStruct((_N_TOK, _D), jnp.float32),
        compiler_params=pltpu.CompilerParams(use_tc_tiling_on_sc=False),
        scratch_types=[
            pltpu.VMEM((b_per_w,), jnp.int32),
            pltpu.VMEM((b_per_w, _D), jnp.float32),
            pltpu.SemaphoreType.DMA,
        ],
    )
    def gather(table_hbm, idx_hbm, out_hbm, idx_v, rows_v, sem):
        wid = lax.axis_index("s")
        base = wid * b_per_w
        r0 = 2 * wid
        r1 = 2 * wid + 1
        pltpu.sync_copy(idx_hbm.at[r0 // 16, r0 % 16], idx_v.at[pl.ds(0, 128)])
        pltpu.sync_copy(idx_hbm.at[r1 // 16, r1 % 16], idx_v.at[pl.ds(128, 128)])
        pltpu.async_copy(table_hbm.at[idx_v], rows_v, sem).wait()
        pltpu.sync_copy(rows_v, out_hbm.at[pl.ds(base, b_per_w)])

    return gather(table, idx)


def kernel(x, embeddings):
    k, z_e = _argmin_call(x, embeddings)
    z_q = _gather_rows(embeddings, k)
    return (z_q, z_e, z_q)


# speedup vs baseline: 1.0092x; 1.0092x over previous
import functools

import jax
import jax.numpy as jnp
from jax import lax
from jax.experimental import pallas as pl
from jax.experimental.pallas import tpu as pltpu
from jax.experimental.pallas import tpu_sc as plsc

_N_TOK = 4096
_N_CLU = 1024
_D = 64
_DP = 128
_TB = 1024


def _argmin_body(x_ref, e_ref, k_ref, ze_ref, et_s, esq_s):
    @pl.when(pl.program_id(0) == 0)
    def _prep():
        e = e_ref[...]
        et = e.T
        et_s[...] = et
        esq_s[...] = jnp.sum(et * et, axis=0, keepdims=True)

    x = x_ref[...]
    dots = lax.dot_general(x, et_s[...], (((1,), (0,)), ((), ())),
                           precision=lax.Precision.HIGHEST,
                           preferred_element_type=jnp.float32)
    scores = esq_s[...] - 2.0 * dots
    k = jnp.argmin(scores, axis=1).astype(jnp.int32)
    k_ref[...] = k.reshape(_TB // 128, 128)
    ze_ref[...] = x


def _argmin_call(x, embeddings):
    grid = _N_TOK // _TB
    return pl.pallas_call(
        _argmin_body,
        grid=(grid,),
        in_specs=[
            pl.BlockSpec((_TB, _D), lambda i: (i, 0)),
            pl.BlockSpec((_N_CLU, _D), lambda i: (0, 0)),
        ],
        out_specs=[
            pl.BlockSpec((_TB // 128, 128), lambda i: (i, 0)),
            pl.BlockSpec((_TB, _D), lambda i: (i, 0)),
        ],
        out_shape=[
            jax.ShapeDtypeStruct((_N_TOK // 128, 128), jnp.int32),
            jax.ShapeDtypeStruct((_N_TOK, _D), jnp.float32),
        ],
        scratch_shapes=[
            pltpu.VMEM((_D, _N_CLU), jnp.float32),
            pltpu.VMEM((1, _N_CLU), jnp.float32),
        ],
    )(x, embeddings)


@functools.partial(jax.jit, static_argnames=())
def _gather_rows(table, idx):
    info = plsc.get_sparse_core_info()
    nw = info.num_subcores
    b_per_w = _N_TOK // nw
    mesh = plsc.VectorSubcoreMesh(core_axis_name="c", subcore_axis_name="s",
                                  num_cores=1)

    @functools.partial(
        pl.kernel, mesh=mesh,
        out_type=jax.ShapeDtypeStruct((_N_TOK, _D), jnp.float32),
        compiler_params=pltpu.CompilerParams(use_tc_tiling_on_sc=False),
        scratch_types=[
            pltpu.VMEM((b_per_w,), jnp.int32),
            pltpu.VMEM((b_per_w, _D), jnp.float32),
            pltpu.SemaphoreType.DMA,
        ],
    )
    def gather(table_hbm, idx_hbm, out_hbm, idx_v, rows_v, sem):
        wid = lax.axis_index("s")
        base = wid * b_per_w
        pltpu.sync_copy(idx_hbm.at[2 * wid], idx_v.at[pl.ds(0, 128)])
        pltpu.sync_copy(idx_hbm.at[2 * wid + 1], idx_v.at[pl.ds(128, 128)])
        pltpu.async_copy(table_hbm.at[idx_v], rows_v, sem).wait()
        pltpu.sync_copy(rows_v, out_hbm.at[pl.ds(base, b_per_w)])

    return gather(table, idx)


def kernel(x, embeddings):
    k, z_e = _argmin_call(x, embeddings)
    z_q = _gather_rows(embeddings, k)
    return (z_q, z_e, z_q)


# speedup vs baseline: 1.0118x; 1.0026x over previous
import functools

import jax
import jax.numpy as jnp
from jax import lax
from jax.experimental import pallas as pl
from jax.experimental.pallas import tpu as pltpu
from jax.experimental.pallas import tpu_sc as plsc

_N_TOK = 4096
_N_CLU = 1024
_D = 64
_TB = 1024


def _argmin_body(x_ref, e_ref, k_ref, ze_ref, et_s, esq_s):
    @pl.when(pl.program_id(0) == 0)
    def _prep():
        et = e_ref[...].T
        et_s[...] = et
        esq_s[...] = jnp.sum(et * et, axis=0, keepdims=True)

    x = x_ref[...]
    dots = lax.dot_general(x, et_s[...], (((1,), (0,)), ((), ())),
                           precision=lax.Precision.HIGHEST,
                           preferred_element_type=jnp.float32)
    scores = esq_s[...] - 2.0 * dots
    k = jnp.argmin(scores, axis=1).astype(jnp.int32)
    k_ref[...] = k.reshape(_TB // 128, 128)
    ze_ref[...] = x


def _argmin_call(x, embeddings):
    grid = _N_TOK // _TB
    return pl.pallas_call(
        _argmin_body,
        grid=(grid,),
        in_specs=[
            pl.BlockSpec((_TB, _D), lambda i: (i, 0)),
            pl.BlockSpec((_N_CLU, _D), lambda i: (0, 0)),
        ],
        out_specs=[
            pl.BlockSpec((_TB // 128, 128), lambda i: (i, 0)),
            pl.BlockSpec((_TB, _D), lambda i: (i, 0)),
        ],
        out_shape=[
            jax.ShapeDtypeStruct((_N_TOK // 128, 128), jnp.int32),
            jax.ShapeDtypeStruct((_N_TOK, _D), jnp.float32),
        ],
        scratch_shapes=[
            pltpu.VMEM((_D, _N_CLU), jnp.float32),
            pltpu.VMEM((1, _N_CLU), jnp.float32),
        ],
    )(x, embeddings)


def _gather_rows(table, idx):
    info = plsc.get_sparse_core_info()
    nw = info.num_subcores
    b_per_w = _N_TOK // nw
    mesh = plsc.VectorSubcoreMesh(core_axis_name="c", subcore_axis_name="s",
                                  num_cores=1)

    @functools.partial(
        pl.kernel, mesh=mesh,
        out_type=jax.ShapeDtypeStruct((_N_TOK, _D), jnp.float32),
        compiler_params=pltpu.CompilerParams(use_tc_tiling_on_sc=False),
        scratch_types=[
            pltpu.VMEM((b_per_w,), jnp.int32),
            pltpu.VMEM((b_per_w, _D), jnp.float32),
            pltpu.SemaphoreType.DMA,
        ],
    )
    def gather(table_hbm, idx_hbm, out_hbm, idx_v, rows_v, sem):
        wid = lax.axis_index("s")
        base = wid * b_per_w
        pltpu.sync_copy(idx_hbm.at[2 * wid], idx_v.at[pl.ds(0, 128)])
        pltpu.sync_copy(idx_hbm.at[2 * wid + 1], idx_v.at[pl.ds(128, 128)])
        pltpu.async_copy(table_hbm.at[idx_v], rows_v, sem).wait()
        pltpu.sync_copy(rows_v, out_hbm.at[pl.ds(base, b_per_w)])

    return gather(table, idx)


def kernel(x, embeddings):
    k, z_e = _argmin_call(x, embeddings)
    z_q = _gather_rows(embeddings, k)
    return (z_q, z_e, z_q)
